# baseline (speedup 1.0000x reference)
import jax
import jax.numpy as jnp
from jax import lax
from jax.experimental import pallas as pl
from jax.experimental.pallas import tpu as pltpu
from jax.experimental.pallas import tpu_sc as plsc

GAMMA = 12.0
EMB_RANGE = 0.21875
PI = 3.141592653589793
PHASE_SCALE = PI / EMB_RANGE

B = 4096
D = 64
ED = 128
NREL = 1000
NREL_PAD = 1024
NC, NS = 2, 16
NW = NC * NS
BPW = B // NW
VL = 16


def _phase_table_body(r_ref, o_ref):
    ph = r_ref[...] * PHASE_SCALE
    o_ref[pl.ds(0, NREL), pl.ds(0, D)] = jnp.cos(ph)
    o_ref[pl.ds(0, NREL), pl.ds(D, D)] = jnp.sin(ph)


def _gather_body(ent_hbm, cs_hbm, hidx_hbm, tidx_hbm, ridx_hbm, sq_hbm,
                 hidx_v, tidx_v, ridx_v, hbuf, tbuf, rbuf, sqbuf,
                 s1, s2, s3, s4):
    wid = lax.axis_index("s") * NC + lax.axis_index("c")
    base = wid * BPW
    pltpu.sync_copy(hidx_hbm.at[pl.ds(base, BPW)], hidx_v)
    pltpu.sync_copy(tidx_hbm.at[pl.ds(base, BPW)], tidx_v)
    pltpu.sync_copy(ridx_hbm.at[pl.ds(base, BPW)], ridx_v)
    gh = pltpu.async_copy(ent_hbm.at[hidx_v], hbuf, s1)
    gt = pltpu.async_copy(ent_hbm.at[tidx_v], tbuf, s2)
    gr = pltpu.async_copy(cs_hbm.at[ridx_v], rbuf, s3)
    gh.wait()
    gt.wait()
    gr.wait()

    iota = lax.iota(jnp.int32, VL)
    jrows = [j * VL + iota for j in range(D // VL)]
    zero = iota * 0

    def sample_body(i, carry):
        col = zero + i
        for j in range(D // VL):
            h_re = hbuf[i, pl.ds(j * VL, VL)]
            h_im = hbuf[i, pl.ds(D + j * VL, VL)]
            t_re = tbuf[i, pl.ds(j * VL, VL)]
            t_im = tbuf[i, pl.ds(D + j * VL, VL)]
            c_re = rbuf[i, pl.ds(j * VL, VL)]
            c_im = rbuf[i, pl.ds(D + j * VL, VL)]
            re_s = h_re * c_re - h_im * c_im - t_re
            im_s = h_re * c_im + h_im * c_re - t_im
            plsc.store_scatter(sqbuf, [jrows[j], col],
                               re_s * re_s + im_s * im_s)
        return carry

    lax.fori_loop(0, BPW, sample_body, 0)
    wq = pltpu.async_copy(sqbuf, sq_hbm.at[pl.ds(wid * D, D)], s4)
    wq.wait()


def _score_body(q_ref, o_ref):
    mag = jnp.sqrt(q_ref[...])
    nw = o_ref.shape[0]
    s = jnp.sum(mag.reshape(nw, D, ED), axis=1)
    o_ref[...] = GAMMA - s


def kernel(sample, entity_embedding, relation_embedding):
    sample = sample.astype(jnp.int32)
    hidx = sample[:, 0]
    tidx = sample[:, 2]
    ridx = sample[:, 1]
    f32 = jnp.float32

    cossin = pl.pallas_call(
        _phase_table_body,
        out_shape=jax.ShapeDtypeStruct((NREL_PAD, ED), f32),
    )(relation_embedding)

    mesh = plsc.VectorSubcoreMesh(core_axis_name="c", subcore_axis_name="s")
    gather = pl.kernel(
        _gather_body,
        out_type=jax.ShapeDtypeStruct((NW * D, ED), f32),
        mesh=mesh,
        compiler_params=pltpu.CompilerParams(needs_layout_passes=False),
        scratch_types=[
            pltpu.VMEM((BPW,), jnp.int32),
            pltpu.VMEM((BPW,), jnp.int32),
            pltpu.VMEM((BPW,), jnp.int32),
            pltpu.VMEM((BPW, ED), f32),
            pltpu.VMEM((BPW, ED), f32),
            pltpu.VMEM((BPW, ED), f32),
            pltpu.VMEM((D, ED), f32),
            pltpu.SemaphoreType.DMA,
            pltpu.SemaphoreType.DMA,
            pltpu.SemaphoreType.DMA,
            pltpu.SemaphoreType.DMA,
        ],
    )
    sq = gather(entity_embedding, cossin, hidx, tidx, ridx)

    nblk = 2
    blk = NW * D // nblk
    score = pl.pallas_call(
        _score_body,
        out_shape=jax.ShapeDtypeStruct((B // ED, ED), f32),
        grid=(nblk,),
        in_specs=[pl.BlockSpec((blk, ED), lambda i: (i, 0))],
        out_specs=pl.BlockSpec((blk // D, ED), lambda i: (i, 0)),
    )(sq)
    return score.reshape(B, 1)


# speedup vs baseline: 1.1058x; 1.1058x over previous
import jax
import jax.numpy as jnp
from jax import lax
from jax.experimental import pallas as pl
from jax.experimental.pallas import tpu as pltpu
from jax.experimental.pallas import tpu_sc as plsc

GAMMA = 12.0
EMB_RANGE = 0.21875
PI = 3.141592653589793
PHASE_SCALE = PI / EMB_RANGE

B = 4096
D = 64
ED = 128
NREL = 1000
NREL_PAD = 1024
NC, NS = 2, 16
NW = NC * NS
BPW = B // NW
VL = 16


def _phase_table_body(r_ref, o_ref):
    ph = r_ref[...] * PHASE_SCALE
    o_ref[pl.ds(0, NREL), pl.ds(0, D)] = jnp.cos(ph)
    o_ref[pl.ds(0, NREL), pl.ds(D, D)] = jnp.sin(ph)


def _gather_body(ent_hbm, cs_hbm, hidx_hbm, tidx_hbm, ridx_hbm, sq_hbm,
                 hidx_v, tidx_v, ridx_v, hbuf, tbuf, rbuf, sqbuf,
                 s1, s2, s3, s4):
    wid = lax.axis_index("s") * NC + lax.axis_index("c")
    base = wid * BPW
    pltpu.sync_copy(hidx_hbm.at[pl.ds(base, BPW)], hidx_v)
    pltpu.sync_copy(tidx_hbm.at[pl.ds(base, BPW)], tidx_v)
    pltpu.sync_copy(ridx_hbm.at[pl.ds(base, BPW)], ridx_v)
    gh = pltpu.async_copy(ent_hbm.at[hidx_v], hbuf, s1)
    gt = pltpu.async_copy(ent_hbm.at[tidx_v], tbuf, s2)
    gr = pltpu.async_copy(cs_hbm.at[ridx_v], rbuf, s3)
    gh.wait()
    gt.wait()
    gr.wait()

    iota = lax.iota(jnp.int32, VL)
    jrows = [j * VL + iota for j in range(D // VL)]
    zero = iota * 0

    @plsc.parallel_loop(0, BPW, unroll=4)
    def sample_body(i):
        col = zero + i
        for j in range(D // VL):
            h_re = hbuf[i, pl.ds(j * VL, VL)]
            h_im = hbuf[i, pl.ds(D + j * VL, VL)]
            t_re = tbuf[i, pl.ds(j * VL, VL)]
            t_im = tbuf[i, pl.ds(D + j * VL, VL)]
            c_re = rbuf[i, pl.ds(j * VL, VL)]
            c_im = rbuf[i, pl.ds(D + j * VL, VL)]
            re_s = h_re * c_re - h_im * c_im - t_re
            im_s = h_re * c_im + h_im * c_re - t_im
            plsc.store_scatter(sqbuf, [jrows[j], col],
                               re_s * re_s + im_s * im_s)
    wq = pltpu.async_copy(sqbuf, sq_hbm.at[pl.ds(wid * D, D)], s4)
    wq.wait()


def _score_body(q_ref, o_ref):
    mag = jnp.sqrt(q_ref[...])
    nw = o_ref.shape[0]
    s = jnp.sum(mag.reshape(nw, D, ED), axis=1)
    o_ref[...] = GAMMA - s


def kernel(sample, entity_embedding, relation_embedding):
    sample = sample.astype(jnp.int32)
    hidx = sample[:, 0]
    tidx = sample[:, 2]
    ridx = sample[:, 1]
    f32 = jnp.float32

    cossin = pl.pallas_call(
        _phase_table_body,
        out_shape=jax.ShapeDtypeStruct((NREL_PAD, ED), f32),
    )(relation_embedding)

    mesh = plsc.VectorSubcoreMesh(core_axis_name="c", subcore_axis_name="s")
    gather = pl.kernel(
        _gather_body,
        out_type=jax.ShapeDtypeStruct((NW * D, ED), f32),
        mesh=mesh,
        compiler_params=pltpu.CompilerParams(needs_layout_passes=False),
        scratch_types=[
            pltpu.VMEM((BPW,), jnp.int32),
            pltpu.VMEM((BPW,), jnp.int32),
            pltpu.VMEM((BPW,), jnp.int32),
            pltpu.VMEM((BPW, ED), f32),
            pltpu.VMEM((BPW, ED), f32),
            pltpu.VMEM((BPW, ED), f32),
            pltpu.VMEM((D, ED), f32),
            pltpu.SemaphoreType.DMA,
            pltpu.SemaphoreType.DMA,
            pltpu.SemaphoreType.DMA,
            pltpu.SemaphoreType.DMA,
        ],
    )
    sq = gather(entity_embedding, cossin, hidx, tidx, ridx)

    nblk = 2
    blk = NW * D // nblk
    score = pl.pallas_call(
        _score_body,
        out_shape=jax.ShapeDtypeStruct((B // ED, ED), f32),
        grid=(nblk,),
        in_specs=[pl.BlockSpec((blk, ED), lambda i: (i, 0))],
        out_specs=pl.BlockSpec((blk // D, ED), lambda i: (i, 0)),
    )(sq)
    return score.reshape(B, 1)
